# Initial kernel scaffold; baseline (speedup 1.0000x reference)
import jax
import jax.numpy as jnp
from jax.experimental import pallas as pl


def kernel(x, W_train, W_pre):
    raise NotImplementedError("write your pallas kernel here")


# speedup vs baseline: 2.0800x; 2.0800x over previous
import functools

import jax
import jax.numpy as jnp
from jax import lax
from jax.experimental import pallas as pl
from jax.experimental.pallas import tpu as pltpu
from jax.experimental.pallas import tpu_sc as plsc


def _table_sum_padded(w_a, w_b, d_pad):
    v, d = w_a.shape
    blk = 2000
    grid = (v + blk - 1) // blk

    def body(a_ref, b_ref, o_ref):
        s = a_ref[...] + b_ref[...]
        o_ref[...] = jnp.concatenate(
            [s, jnp.zeros((s.shape[0], d_pad - d), jnp.float32)], axis=1)

    return pl.pallas_call(
        body,
        grid=(grid,),
        in_specs=[
            pl.BlockSpec((blk, d), lambda i: (i, 0)),
            pl.BlockSpec((blk, d), lambda i: (i, 0)),
        ],
        out_specs=pl.BlockSpec((blk, d_pad), lambda i: (i, 0)),
        out_shape=jax.ShapeDtypeStruct((v, d_pad), jnp.float32),
    )(w_a, w_b)


@functools.lru_cache(maxsize=None)
def _make_gather(n, v, d, d_pad, nc, ns):
    nw = nc * ns
    ch = 128
    assert n % (nw * ch) == 0
    per_w = n // nw
    n_ch = per_w // ch
    d_lo = (d // 128) * 128
    d_hi = d - d_lo
    mesh = plsc.VectorSubcoreMesh(
        core_axis_name="c", subcore_axis_name="s", num_cores=nc)

    @functools.partial(
        pl.kernel,
        mesh=mesh,
        out_type=jax.ShapeDtypeStruct((n, d), jnp.float32),
        scratch_types=[
            pltpu.VMEM((n_ch, ch), jnp.int32),
            pltpu.VMEM((ch, d_pad), jnp.float32),
            pltpu.VMEM((ch, d_hi), jnp.float32),
            pltpu.SemaphoreType.DMA,
        ],
    )
    def gather_kernel(table_hbm, idx_hbm, out_hbm, idx_v, rows_v, tail_v, sem):
        wid = lax.axis_index("s") * nc + lax.axis_index("c")
        base = wid * per_w
        pltpu.sync_copy(idx_hbm.at[pl.ds(wid * n_ch, n_ch)], idx_v)

        def body(j, carry):
            pltpu.async_copy(table_hbm.at[idx_v.at[j]], rows_v, sem).wait()

            def rcopy(r, c2):
                tail_v[r, pl.ds(0, 16)] = rows_v[r, pl.ds(d_lo, 16)]
                tail_v[r, pl.ds(16, 16)] = rows_v[r, pl.ds(d_lo + 16, 16)]
                tail_v[r, pl.ds(d_hi - 16, 16)] = (
                    rows_v[r, pl.ds(d - 16, 16)])
                return c2

            lax.fori_loop(0, ch, rcopy, 0)
            rows = pl.ds(base + j * ch, ch)
            pltpu.sync_copy(rows_v.at[:, pl.ds(0, d_lo)],
                            out_hbm.at[rows, pl.ds(0, d_lo)])
            pltpu.sync_copy(tail_v, out_hbm.at[rows, pl.ds(d_lo, d_hi)])
            return carry

        lax.fori_loop(0, n_ch, body, 0)

    def run(table, idx_flat):
        return gather_kernel(table, idx_flat.reshape(n // ch, ch))

    return run


def kernel(x, W_train, W_pre):
    b, l = x.shape
    v, d = W_train.shape
    n = b * l
    d_pad = ((d + 127) // 128) * 128
    w_sum = _table_sum_padded(W_train, W_pre, d_pad)
    info = plsc.get_sparse_core_info()
    gather = _make_gather(n, v, d, d_pad, info.num_cores, info.num_subcores)
    out = gather(w_sum, x.reshape(n).astype(jnp.int32))
    return out.reshape(b, l, d)
